# baseline (speedup 1.0000x reference)
import functools

import jax
import jax.numpy as jnp
from jax import lax
from jax.experimental import pallas as pl
from jax.experimental.pallas import tpu as pltpu
from jax.experimental.pallas import tpu_sc as plsc

NC = 2
NS = 16
CHUNK = 128
RB = 2000

F32 = jnp.float32


KS = 3
ZR = 64


def _zero_acc(acc, zero_v, semg, s, d, rows_w, zcopies):
    zvec = jnp.zeros((16,), F32)

    def zrow(i, carry):
        for j in range(d // 16):
            zero_v[i, pl.ds(j * 16, 16)] = zvec
        return carry

    lax.fori_loop(0, ZR, zrow, 0)

    descs = [pltpu.async_copy(
        zero_v, acc.at[pl.ds(s * rows_w + t * ZR, ZR)], semg)
        for t in range(zcopies)]
    for dsc in descs:
        dsc.wait()


def _edge_groups(table, ei_hbm, acc, idxb, rows, semg, sems2, cbase, groups):
    def scat_wait(b):
        for k in range(KS):
            pltpu.make_async_copy(rows.at[b, k], acc.at[idxb.at[b, k, 1]],
                                  sems2[b]).wait()

    def half(g, b, first):
        off = cbase + g * KS

        if not first:
            scat_wait(b)
        pltpu.sync_copy(ei_hbm.at[pl.ds(off, KS)], idxb.at[b])
        descs = [pltpu.async_copy(table.at[idxb.at[b, k, 0]], rows.at[b, k],
                                  semg) for k in range(KS)]
        for dsc in descs:
            dsc.wait()
        for k in range(KS):
            pltpu.async_copy(rows.at[b, k], acc.at[idxb.at[b, k, 1]],
                             sems2[b], add=True)

    half(cbase * 0, 0, True)
    half(cbase * 0 + 1, 1, True)

    def body(g2, carry):
        half(2 * g2, 0, False)
        half(2 * g2 + 1, 1, False)
        return carry

    lax.fori_loop(1, groups // 2, body, 0)
    scat_wait(0)
    scat_wait(1)


@functools.lru_cache(maxsize=None)
def _make_sc_agg(n_rows, d, n_pad, e_pad):
    nw = NC * NS
    e_w = e_pad // nw
    chunks = e_w // CHUNK
    rows_w = n_pad // NS
    zcopies = rows_w // ZR
    mesh = plsc.VectorSubcoreMesh(core_axis_name="c", subcore_axis_name="s",
                                  num_cores=NC, num_subcores=NS)

    groups = chunks // KS

    @functools.partial(
        pl.kernel,
        out_type=jax.ShapeDtypeStruct((NC, n_pad, d), F32),
        mesh=mesh,
        scratch_types=[
            pltpu.VMEM_SHARED((n_pad, d), F32),
            pltpu.VMEM((2, KS, 2, CHUNK), jnp.int32),
            pltpu.VMEM((2, KS, CHUNK, d), F32),
            pltpu.VMEM((ZR, d), F32),
            pltpu.SemaphoreType.DMA,
            pltpu.SemaphoreType.DMA,
            pltpu.SemaphoreType.DMA,
        ],
        compiler_params=pltpu.CompilerParams(use_tc_tiling_on_sc=False),
    )
    def agg(u_hbm, ei_hbm, out_hbm, acc, idxb, rows, zero_v, semg,
            sems_a, sems_b):
        c = lax.axis_index("c")
        s = lax.axis_index("s")
        _zero_acc(acc, zero_v, semg, s, d, rows_w, zcopies)
        plsc.subcore_barrier()
        cbase = (c * NS + s) * chunks
        _edge_groups(u_hbm, ei_hbm, acc, idxb, rows, semg,
                     (sems_a, sems_b), cbase, groups)
        plsc.subcore_barrier()
        pltpu.sync_copy(acc.at[pl.ds(s * rows_w, rows_w)],
                        out_hbm.at[c, pl.ds(s * rows_w, rows_w)])

    return agg


def _sc_agg(u, ei_p, n_pad, e_pad):
    n, d = u.shape
    out = _make_sc_agg(n, d, n_pad, e_pad)(u, ei_p)
    return out[0, :n], out[1, :n]


@functools.lru_cache(maxsize=None)
def _make_sc_agg_fsplit(n_rows, d, n_pad, e_pad):
    e_s = e_pad // NS
    chunks = e_s // CHUNK
    rows_w = n_pad // NS
    zcopies = rows_w // ZR
    mesh = plsc.VectorSubcoreMesh(core_axis_name="c", subcore_axis_name="s",
                                  num_cores=NC, num_subcores=NS)

    groups = chunks // KS

    @functools.partial(
        pl.kernel,
        out_type=jax.ShapeDtypeStruct((NC, n_pad, d), F32),
        mesh=mesh,
        scratch_types=[
            pltpu.VMEM_SHARED((n_pad, d), F32),
            pltpu.VMEM((2, KS, 2, CHUNK), jnp.int32),
            pltpu.VMEM((2, KS, CHUNK, d), F32),
            pltpu.VMEM((ZR, d), F32),
            pltpu.SemaphoreType.DMA,
            pltpu.SemaphoreType.DMA,
            pltpu.SemaphoreType.DMA,
        ],
        compiler_params=pltpu.CompilerParams(use_tc_tiling_on_sc=False),
    )
    def agg(t0_hbm, t1_hbm, ei_hbm, out_hbm, acc, idxb, rows, zero_v, semg,
            sems_a, sems_b):
        c = lax.axis_index("c")
        s = lax.axis_index("s")
        _zero_acc(acc, zero_v, semg, s, d, rows_w, zcopies)
        plsc.subcore_barrier()
        cbase = s * chunks

        @pl.when(c == 0)
        def _():
            _edge_groups(t0_hbm, ei_hbm, acc, idxb, rows, semg,
                         (sems_a, sems_b), cbase, groups)

        @pl.when(c == 1)
        def _():
            _edge_groups(t1_hbm, ei_hbm, acc, idxb, rows, semg,
                         (sems_a, sems_b), cbase, groups)

        plsc.subcore_barrier()
        pltpu.sync_copy(acc.at[pl.ds(s * rows_w, rows_w)],
                        out_hbm.at[c, pl.ds(s * rows_w, rows_w)])

    return agg


def _sc_agg_fs(t0, t1, ei_p, n_pad, e_pad):
    n, d = t0.shape
    out = _make_sc_agg_fsplit(n, d, n_pad, e_pad)(t0, t1, ei_p)
    return out[0, :n], out[1, :n]


HI = lax.Precision.HIGHEST


def _acc_stats(st_ref, u, i):
    rbf = float(u.shape[0])
    m_b = jnp.mean(u, axis=0, keepdims=True)
    du = u - m_b
    m2_b = jnp.sum(du * du, axis=0, keepdims=True)
    pad = jnp.zeros((6, u.shape[1]), F32)

    @pl.when(i == 0)
    def _():
        st_ref[...] = jnp.concatenate([m_b, m2_b, pad], axis=0)

    @pl.when(i > 0)
    def _():
        fi = i.astype(F32)
        mean_run = st_ref[0:1, :]
        m2_run = st_ref[1:2, :]
        delta = m_b - mean_run
        new_mean = mean_run + delta * (1.0 / (fi + 1.0))
        new_m2 = m2_run + m2_b + delta * delta * (rbf * fi / (fi + 1.0))
        st_ref[...] = jnp.concatenate([new_mean, new_m2, pad], axis=0)


def _mlp0(x, aa, ab, t0, t1, wa, ba, wb, bb, fd):
    n = x.shape[0]
    d = wa.shape[1]
    da = aa.shape[1]
    dt = t0.shape[1]

    def body(x_ref, aa_ref, ab_ref, t0_ref, t1_ref, wa_ref, ba_ref, wb_ref,
             bb_ref, o_ref, st_ref):
        i = pl.program_id(0)
        tail = (t0_ref[...] + t1_ref[...])[:, :fd - 2 * da]
        agg = jnp.concatenate([aa_ref[...], ab_ref[...], tail], axis=1)
        h = x_ref[...] + agg
        t = jnp.maximum(jnp.dot(h, wa_ref[...], preferred_element_type=F32)
                        + ba_ref[...], 0.0)
        u = jnp.maximum(jnp.dot(t, wb_ref[...], preferred_element_type=F32)
                        + bb_ref[...], 0.0)
        o_ref[...] = u
        _acc_stats(st_ref, u, i)

    return pl.pallas_call(
        body,
        grid=(n // RB,),
        in_specs=[pl.BlockSpec((RB, fd), lambda i: (i, 0)),
                  pl.BlockSpec((RB, da), lambda i: (i, 0)),
                  pl.BlockSpec((RB, da), lambda i: (i, 0)),
                  pl.BlockSpec((RB, dt), lambda i: (i, 0)),
                  pl.BlockSpec((RB, dt), lambda i: (i, 0)),
                  pl.BlockSpec((fd, d), lambda i: (0, 0)),
                  pl.BlockSpec((1, d), lambda i: (0, 0)),
                  pl.BlockSpec((d, d), lambda i: (0, 0)),
                  pl.BlockSpec((1, d), lambda i: (0, 0))],
        out_specs=[pl.BlockSpec((RB, d), lambda i: (i, 0)),
                   pl.BlockSpec((8, d), lambda i: (0, 0))],
        out_shape=[jax.ShapeDtypeStruct((n, d), F32),
                   jax.ShapeDtypeStruct((8, d), F32)],
    )(x, aa, ab, t0, t1, wa, ba, wb, bb)


def _mlp_fold(u, st, g, b, a0, a1, deg, wa, ba, wb, bb):
    n, d = u.shape

    def body(u_ref, st_ref, g_ref, b_ref, a0_ref, a1_ref, deg_ref,
             wa_ref, ba_ref, wb_ref, bb_ref, o_ref, sto_ref):
        i = pl.program_id(0)
        scale, shift = _bn_scale_shift(st_ref, g_ref, b_ref, n)
        h = ((u_ref[...] + a0_ref[...] + a1_ref[...]) * scale
             + (deg_ref[...] + 1.0) * shift)
        t = jnp.maximum(jnp.dot(h, wa_ref[...], preferred_element_type=F32)
                        + ba_ref[...], 0.0)
        un = jnp.maximum(jnp.dot(t, wb_ref[...], preferred_element_type=F32)
                         + bb_ref[...], 0.0)
        o_ref[...] = un
        _acc_stats(sto_ref, un, i)

    return pl.pallas_call(
        body,
        grid=(n // RB,),
        in_specs=[pl.BlockSpec((RB, d), lambda i: (i, 0)),
                  pl.BlockSpec((8, d), lambda i: (0, 0)),
                  pl.BlockSpec((1, d), lambda i: (0, 0)),
                  pl.BlockSpec((1, d), lambda i: (0, 0)),
                  pl.BlockSpec((RB, d), lambda i: (i, 0)),
                  pl.BlockSpec((RB, d), lambda i: (i, 0)),
                  pl.BlockSpec((RB, 1), lambda i: (i, 0)),
                  pl.BlockSpec((d, d), lambda i: (0, 0)),
                  pl.BlockSpec((1, d), lambda i: (0, 0)),
                  pl.BlockSpec((d, d), lambda i: (0, 0)),
                  pl.BlockSpec((1, d), lambda i: (0, 0))],
        out_specs=[pl.BlockSpec((RB, d), lambda i: (i, 0)),
                   pl.BlockSpec((8, d), lambda i: (0, 0))],
        out_shape=[jax.ShapeDtypeStruct((n, d), F32),
                   jax.ShapeDtypeStruct((8, d), F32)],
    )(u, st, g, b, a0, a1, deg, wa, ba, wb, bb)


def _bn_scale_shift(st_ref, g_ref, b_ref, n):
    m = st_ref[0:1, :]
    v = st_ref[1:2, :] * (1.0 / n)
    scale = g_ref[...] * lax.rsqrt(v + 1e-5)
    shift = b_ref[...] - m * scale
    return scale, shift


def _bn_pool(u, st, g, b, batch2d, nseg):
    n, d = u.shape

    def body(u_ref, st_ref, g_ref, b_ref, bt_ref, o_ref):
        i = pl.program_id(0)
        scale, shift = _bn_scale_shift(st_ref, g_ref, b_ref, n)
        ubn = u_ref[...] * scale + shift
        seg = lax.broadcasted_iota(jnp.int32, (1, nseg), 1)
        oh = (bt_ref[...] == seg).astype(F32)
        blk = lax.dot_general(oh, ubn, (((0,), (0,)), ((), ())),
                              preferred_element_type=F32, precision=HI)

        @pl.when(i == 0)
        def _():
            o_ref[...] = blk

        @pl.when(i > 0)
        def _():
            o_ref[...] += blk

    return pl.pallas_call(
        body,
        grid=(n // RB,),
        in_specs=[pl.BlockSpec((RB, d), lambda i: (i, 0)),
                  pl.BlockSpec((8, d), lambda i: (0, 0)),
                  pl.BlockSpec((1, d), lambda i: (0, 0)),
                  pl.BlockSpec((1, d), lambda i: (0, 0)),
                  pl.BlockSpec((RB, 1), lambda i: (i, 0))],
        out_specs=pl.BlockSpec((nseg, d), lambda i: (0, 0)),
        out_shape=jax.ShapeDtypeStruct((nseg, d), F32),
    )(u, st, g, b, batch2d)


def _head_g(target, w_r2, vocab):
    bsz, seq = target.shape
    ok = w_r2.shape[1]

    def body(t_ref, w_ref, o_ref):
        v = pl.program_id(0)
        mask = (t_ref[...] == v).astype(F32)
        o_ref[0, :, :] = jnp.dot(mask, w_ref[...], preferred_element_type=F32, precision=HI)

    return pl.pallas_call(
        body,
        grid=(vocab,),
        in_specs=[pl.BlockSpec((bsz, seq), lambda v: (0, 0)),
                  pl.BlockSpec((seq, ok), lambda v: (0, 0))],
        out_specs=pl.BlockSpec((1, bsz, ok), lambda v: (v, 0, 0)),
        out_shape=jax.ShapeDtypeStruct((vocab, bsz, ok), F32),
    )(target, w_r2)


def _head_m(t_stack, w3t):
    kk, vocab, hh = t_stack.shape
    on = w3t.shape[1]
    nb = on // 1024

    def body(t_ref, w_ref, o_ref):
        o_ref[0, :, :] = jnp.dot(t_ref[0, :, :], w_ref[...],
                                 preferred_element_type=F32, precision=HI)

    return pl.pallas_call(
        body,
        grid=(kk, nb),
        in_specs=[pl.BlockSpec((1, vocab, hh), lambda k, nc: (k, 0, 0)),
                  pl.BlockSpec((hh, 1024), lambda k, nc: (0, nc))],
        out_specs=pl.BlockSpec((1, vocab, 1024), lambda k, nc: (k, 0, nc)),
        out_shape=jax.ShapeDtypeStruct((kk, vocab, on), F32),
    )(t_stack, w3t)


def _head_final(pool, g_all, m_all, xd_w, xd_b, xt_bias, fc1_w, fc1_b,
                fc2_w, fc2_b, cls_w, cls_b):
    bsz = pool.shape[0]
    ncls = cls_w.shape[1]

    def body(pool_ref, g_ref, m_ref, xdw_ref, xdb_ref, xtb_ref, f1w_ref,
             f1b_ref, f2w_ref, f2b_ref, cw_ref, cb_ref, o_ref):
        xt = jnp.dot(g_ref[...], m_ref[...],
                     preferred_element_type=F32, precision=HI) + xtb_ref[...]
        xd = jnp.maximum(jnp.dot(pool_ref[...], xdw_ref[...],
                                 preferred_element_type=F32, precision=HI) + xdb_ref[...], 0.0)
        xc = jnp.concatenate([xd, xt], axis=1)
        h1 = jnp.maximum(jnp.dot(xc, f1w_ref[...],
                                 preferred_element_type=F32, precision=HI) + f1b_ref[...], 0.0)
        h2 = jnp.maximum(jnp.dot(h1, f2w_ref[...],
                                 preferred_element_type=F32, precision=HI) + f2b_ref[...], 0.0)
        o_ref[...] = jnp.dot(h2, cw_ref[...],
                             preferred_element_type=F32, precision=HI) + cb_ref[...]

    return pl.pallas_call(
        body,
        out_shape=jax.ShapeDtypeStruct((bsz, ncls), F32),
    )(pool, g_all, m_all, xd_w, xd_b, xt_bias, fc1_w, fc1_b, fc2_w, fc2_b,
      cls_w, cls_b)


def kernel(x, edge_index, batch, target, emb_table, gin_params, bn_params,
           fc_params):
    n, fd = x.shape
    e = edge_index.shape[1]
    d = gin_params[0][0].shape[1]
    bsz, seq = target.shape
    vocab, emb = emb_table.shape
    odim = fc_params['fc1_xd_w'].shape[1]

    per_w = -(-e // (NC * NS * CHUNK * KS)) * (CHUNK * KS)
    e_pad = per_w * NC * NS
    n_pad = -(-(n + 1) // (NS * CHUNK)) * (NS * CHUNK)
    src = edge_index[0]
    dst = edge_index[1]
    pad = e_pad - e
    src_p = jnp.concatenate([src, jnp.zeros((pad,), jnp.int32)])
    dst_p = jnp.concatenate([dst, jnp.full((pad,), n, jnp.int32)])
    ei_p = jnp.stack([src_p.reshape(e_pad // CHUNK, CHUNK),
                      dst_p.reshape(e_pad // CHUNK, CHUNK)], axis=1)

    wa0, ba0, wb0, bb0 = gin_params[0]
    da = 32
    dt = 16
    x0 = x[:, :da]
    x1 = x[:, da:2 * da]
    xt = jnp.concatenate(
        [x[:, 2 * da:], jnp.zeros((n, dt - (fd - 2 * da) - 1), F32),
         jnp.ones((n, 1), F32)], axis=1)
    aa, ab = _sc_agg_fs(x0, x1, ei_p, n_pad, e_pad)
    t0, t1 = _sc_agg(xt, ei_p, n_pad, e_pad)
    u_pre, st = _mlp0(x, aa, ab, t0, t1, wa0, ba0.reshape(1, d), wb0,
                      bb0.reshape(1, d), fd)
    deg = (t0 + t1)[:, dt - 1:dt]
    for i in range(1, len(gin_params)):
        g, b = bn_params[i - 1]
        a0, a1 = _sc_agg(u_pre, ei_p, n_pad, e_pad)
        wa, ba, wb, bb = gin_params[i]
        u_pre, st = _mlp_fold(u_pre, st, g.reshape(1, d), b.reshape(1, d),
                              a0, a1, deg, wa, ba.reshape(1, d), wb,
                              bb.reshape(1, d))
    g, b = bn_params[-1]
    pool = _bn_pool(u_pre, st, g.reshape(1, d), b.reshape(1, d),
                    batch.reshape(n, 1), bsz)

    conv_w = fc_params['conv_w']
    osz, _, ksz = conv_w.shape
    hout = emb - ksz + 1
    w_r2 = conv_w.transpose(1, 2, 0).reshape(seq, ksz * osz)
    w3t = (fc_params['fc1_xt_w'].reshape(osz, hout, odim)
           .transpose(1, 0, 2).reshape(hout, osz * odim))
    t_stack = jnp.stack([emb_table[:, k:k + hout] for k in range(ksz)])
    xt_bias = (fc_params['conv_b'] @ fc_params['fc1_xt_w']
               .reshape(osz, hout, odim).sum(axis=1)
               + fc_params['fc1_xt_b']).reshape(1, odim)

    gt = _head_g(target, w_r2, vocab)
    g_all = gt.transpose(1, 0, 2).reshape(bsz, vocab * ksz * osz)
    m = _head_m(t_stack, w3t)
    m_all = (m.reshape(ksz, vocab, osz, odim).transpose(1, 0, 2, 3)
             .reshape(vocab * ksz * osz, odim))

    return _head_final(
        pool, g_all, m_all,
        fc_params['fc1_xd_w'], fc_params['fc1_xd_b'].reshape(1, odim),
        xt_bias,
        fc_params['fc1_w'], fc_params['fc1_b'].reshape(1, -1),
        fc_params['fc2_w'], fc_params['fc2_b'].reshape(1, -1),
        fc_params['cls_w'], fc_params['cls_b'].reshape(1, -1))


# speedup vs baseline: 1.1636x; 1.1636x over previous
import functools

import jax
import jax.numpy as jnp
from jax import lax
from jax.experimental import pallas as pl
from jax.experimental.pallas import tpu as pltpu
from jax.experimental.pallas import tpu_sc as plsc

NC = 2
NS = 16
CHUNK = 128
RB = 2000

F32 = jnp.float32


KS = 2
ZR = 64


def _zero_acc(acc, zero_v, semg, s, d, rows_w, zcopies):
    zvec = jnp.zeros((16,), F32)

    def zrow(i, carry):
        for j in range(d // 16):
            zero_v[i, pl.ds(j * 16, 16)] = zvec
        return carry

    lax.fori_loop(0, ZR, zrow, 0)

    descs = [pltpu.async_copy(
        zero_v, acc.at[pl.ds(s * rows_w + t * ZR, ZR)], semg)
        for t in range(zcopies)]
    for dsc in descs:
        dsc.wait()


def _edge_groups(table, ei_hbm, acc, idxb, rows, semg, sems2, cbase, groups):
    def scat_wait(b):
        for k in range(KS):
            pltpu.make_async_copy(rows.at[b, k], acc.at[idxb.at[b, k, 1]],
                                  sems2[b]).wait()

    def half(g, b, first):
        off = cbase + g * KS

        if not first:
            scat_wait(b)
        pltpu.sync_copy(ei_hbm.at[pl.ds(off, KS)], idxb.at[b])
        descs = [pltpu.async_copy(table.at[idxb.at[b, k, 0]], rows.at[b, k],
                                  semg) for k in range(KS)]
        for dsc in descs:
            dsc.wait()
        for k in range(KS):
            pltpu.async_copy(rows.at[b, k], acc.at[idxb.at[b, k, 1]],
                             sems2[b], add=True)

    half(cbase * 0, 0, True)
    half(cbase * 0 + 1, 1, True)

    def body(g2, carry):
        half(2 * g2, 0, False)
        half(2 * g2 + 1, 1, False)
        return carry

    lax.fori_loop(1, groups // 2, body, 0)
    scat_wait(0)
    scat_wait(1)


@functools.lru_cache(maxsize=None)
def _make_sc_agg(n_rows, d, n_pad, e_pad):
    nw = NC * NS
    e_w = e_pad // nw
    chunks = e_w // CHUNK
    rows_w = n_pad // NS
    zcopies = rows_w // ZR
    mesh = plsc.VectorSubcoreMesh(core_axis_name="c", subcore_axis_name="s",
                                  num_cores=NC, num_subcores=NS)

    groups = chunks // KS

    @functools.partial(
        pl.kernel,
        out_type=jax.ShapeDtypeStruct((NC, n_pad, d), F32),
        mesh=mesh,
        scratch_types=[
            pltpu.VMEM_SHARED((n_pad, d), F32),
            pltpu.VMEM((2, KS, 2, CHUNK), jnp.int32),
            pltpu.VMEM((2, KS, CHUNK, d), F32),
            pltpu.VMEM((ZR, d), F32),
            pltpu.SemaphoreType.DMA,
            pltpu.SemaphoreType.DMA,
            pltpu.SemaphoreType.DMA,
        ],
        compiler_params=pltpu.CompilerParams(use_tc_tiling_on_sc=False),
    )
    def agg(u_hbm, ei_hbm, out_hbm, acc, idxb, rows, zero_v, semg,
            sems_a, sems_b):
        c = lax.axis_index("c")
        s = lax.axis_index("s")
        _zero_acc(acc, zero_v, semg, s, d, rows_w, zcopies)
        plsc.subcore_barrier()
        cbase = (c * NS + s) * chunks
        _edge_groups(u_hbm, ei_hbm, acc, idxb, rows, semg,
                     (sems_a, sems_b), cbase, groups)
        plsc.subcore_barrier()
        pltpu.sync_copy(acc.at[pl.ds(s * rows_w, rows_w)],
                        out_hbm.at[c, pl.ds(s * rows_w, rows_w)])

    return agg


def _sc_agg(u, ei_p, n_pad, e_pad):
    n, d = u.shape
    out = _make_sc_agg(n, d, n_pad, e_pad)(u, ei_p)
    return out[0, :n], out[1, :n]


@functools.lru_cache(maxsize=None)
def _make_sc_agg_fsplit(n_rows, d, n_pad, e_pad):
    e_s = e_pad // NS
    chunks = e_s // CHUNK
    rows_w = n_pad // NS
    zcopies = rows_w // ZR
    mesh = plsc.VectorSubcoreMesh(core_axis_name="c", subcore_axis_name="s",
                                  num_cores=NC, num_subcores=NS)

    groups = chunks // KS

    @functools.partial(
        pl.kernel,
        out_type=jax.ShapeDtypeStruct((NC, n_pad, d), F32),
        mesh=mesh,
        scratch_types=[
            pltpu.VMEM_SHARED((n_pad, d), F32),
            pltpu.VMEM((2, KS, 2, CHUNK), jnp.int32),
            pltpu.VMEM((2, KS, CHUNK, d), F32),
            pltpu.VMEM((ZR, d), F32),
            pltpu.SemaphoreType.DMA,
            pltpu.SemaphoreType.DMA,
            pltpu.SemaphoreType.DMA,
        ],
        compiler_params=pltpu.CompilerParams(use_tc_tiling_on_sc=False),
    )
    def agg(t0_hbm, t1_hbm, ei_hbm, out_hbm, acc, idxb, rows, zero_v, semg,
            sems_a, sems_b):
        c = lax.axis_index("c")
        s = lax.axis_index("s")
        _zero_acc(acc, zero_v, semg, s, d, rows_w, zcopies)
        plsc.subcore_barrier()
        cbase = s * chunks

        @pl.when(c == 0)
        def _():
            _edge_groups(t0_hbm, ei_hbm, acc, idxb, rows, semg,
                         (sems_a, sems_b), cbase, groups)

        @pl.when(c == 1)
        def _():
            _edge_groups(t1_hbm, ei_hbm, acc, idxb, rows, semg,
                         (sems_a, sems_b), cbase, groups)

        plsc.subcore_barrier()
        pltpu.sync_copy(acc.at[pl.ds(s * rows_w, rows_w)],
                        out_hbm.at[c, pl.ds(s * rows_w, rows_w)])

    return agg


def _sc_agg_fs(t0, t1, ei_p, n_pad, e_pad):
    n, d = t0.shape
    out = _make_sc_agg_fsplit(n, d, n_pad, e_pad)(t0, t1, ei_p)
    return out[0, :n], out[1, :n]


HI = lax.Precision.HIGHEST


def _acc_stats(st_ref, u, i):
    rbf = float(u.shape[0])
    m_b = jnp.mean(u, axis=0, keepdims=True)
    du = u - m_b
    m2_b = jnp.sum(du * du, axis=0, keepdims=True)
    pad = jnp.zeros((6, u.shape[1]), F32)

    @pl.when(i == 0)
    def _():
        st_ref[...] = jnp.concatenate([m_b, m2_b, pad], axis=0)

    @pl.when(i > 0)
    def _():
        fi = i.astype(F32)
        mean_run = st_ref[0:1, :]
        m2_run = st_ref[1:2, :]
        delta = m_b - mean_run
        new_mean = mean_run + delta * (1.0 / (fi + 1.0))
        new_m2 = m2_run + m2_b + delta * delta * (rbf * fi / (fi + 1.0))
        st_ref[...] = jnp.concatenate([new_mean, new_m2, pad], axis=0)


def _mlp0(x, aa, ab, t0, t1, wa, ba, wb, bb, fd):
    n = x.shape[0]
    d = wa.shape[1]
    da = aa.shape[1]
    dt = t0.shape[1]

    def body(x_ref, aa_ref, ab_ref, t0_ref, t1_ref, wa_ref, ba_ref, wb_ref,
             bb_ref, o_ref, st_ref):
        i = pl.program_id(0)
        tail = (t0_ref[...] + t1_ref[...])[:, :fd - 2 * da]
        agg = jnp.concatenate([aa_ref[...], ab_ref[...], tail], axis=1)
        h = x_ref[...] + agg
        t = jnp.maximum(jnp.dot(h, wa_ref[...], preferred_element_type=F32)
                        + ba_ref[...], 0.0)
        u = jnp.maximum(jnp.dot(t, wb_ref[...], preferred_element_type=F32)
                        + bb_ref[...], 0.0)
        o_ref[...] = u
        _acc_stats(st_ref, u, i)

    return pl.pallas_call(
        body,
        grid=(n // RB,),
        in_specs=[pl.BlockSpec((RB, fd), lambda i: (i, 0)),
                  pl.BlockSpec((RB, da), lambda i: (i, 0)),
                  pl.BlockSpec((RB, da), lambda i: (i, 0)),
                  pl.BlockSpec((RB, dt), lambda i: (i, 0)),
                  pl.BlockSpec((RB, dt), lambda i: (i, 0)),
                  pl.BlockSpec((fd, d), lambda i: (0, 0)),
                  pl.BlockSpec((1, d), lambda i: (0, 0)),
                  pl.BlockSpec((d, d), lambda i: (0, 0)),
                  pl.BlockSpec((1, d), lambda i: (0, 0))],
        out_specs=[pl.BlockSpec((RB, d), lambda i: (i, 0)),
                   pl.BlockSpec((8, d), lambda i: (0, 0))],
        out_shape=[jax.ShapeDtypeStruct((n, d), F32),
                   jax.ShapeDtypeStruct((8, d), F32)],
    )(x, aa, ab, t0, t1, wa, ba, wb, bb)


def _mlp_fold(u, st, g, b, a0, a1, deg, wa, ba, wb, bb):
    n, d = u.shape

    def body(u_ref, st_ref, g_ref, b_ref, a0_ref, a1_ref, deg_ref,
             wa_ref, ba_ref, wb_ref, bb_ref, o_ref, sto_ref):
        i = pl.program_id(0)
        scale, shift = _bn_scale_shift(st_ref, g_ref, b_ref, n)
        h = ((u_ref[...] + a0_ref[...] + a1_ref[...]) * scale
             + (deg_ref[...] + 1.0) * shift)
        t = jnp.maximum(jnp.dot(h, wa_ref[...], preferred_element_type=F32)
                        + ba_ref[...], 0.0)
        un = jnp.maximum(jnp.dot(t, wb_ref[...], preferred_element_type=F32)
                         + bb_ref[...], 0.0)
        o_ref[...] = un
        _acc_stats(sto_ref, un, i)

    return pl.pallas_call(
        body,
        grid=(n // RB,),
        in_specs=[pl.BlockSpec((RB, d), lambda i: (i, 0)),
                  pl.BlockSpec((8, d), lambda i: (0, 0)),
                  pl.BlockSpec((1, d), lambda i: (0, 0)),
                  pl.BlockSpec((1, d), lambda i: (0, 0)),
                  pl.BlockSpec((RB, d), lambda i: (i, 0)),
                  pl.BlockSpec((RB, d), lambda i: (i, 0)),
                  pl.BlockSpec((RB, 1), lambda i: (i, 0)),
                  pl.BlockSpec((d, d), lambda i: (0, 0)),
                  pl.BlockSpec((1, d), lambda i: (0, 0)),
                  pl.BlockSpec((d, d), lambda i: (0, 0)),
                  pl.BlockSpec((1, d), lambda i: (0, 0))],
        out_specs=[pl.BlockSpec((RB, d), lambda i: (i, 0)),
                   pl.BlockSpec((8, d), lambda i: (0, 0))],
        out_shape=[jax.ShapeDtypeStruct((n, d), F32),
                   jax.ShapeDtypeStruct((8, d), F32)],
    )(u, st, g, b, a0, a1, deg, wa, ba, wb, bb)


def _bn_scale_shift(st_ref, g_ref, b_ref, n):
    m = st_ref[0:1, :]
    v = st_ref[1:2, :] * (1.0 / n)
    scale = g_ref[...] * lax.rsqrt(v + 1e-5)
    shift = b_ref[...] - m * scale
    return scale, shift


def _bn_pool(u, st, g, b, batch2d, nseg):
    n, d = u.shape

    def body(u_ref, st_ref, g_ref, b_ref, bt_ref, o_ref):
        i = pl.program_id(0)
        scale, shift = _bn_scale_shift(st_ref, g_ref, b_ref, n)
        ubn = u_ref[...] * scale + shift
        seg = lax.broadcasted_iota(jnp.int32, (1, nseg), 1)
        oh = (bt_ref[...] == seg).astype(F32)
        blk = lax.dot_general(oh, ubn, (((0,), (0,)), ((), ())),
                              preferred_element_type=F32, precision=HI)

        @pl.when(i == 0)
        def _():
            o_ref[...] = blk

        @pl.when(i > 0)
        def _():
            o_ref[...] += blk

    return pl.pallas_call(
        body,
        grid=(n // RB,),
        in_specs=[pl.BlockSpec((RB, d), lambda i: (i, 0)),
                  pl.BlockSpec((8, d), lambda i: (0, 0)),
                  pl.BlockSpec((1, d), lambda i: (0, 0)),
                  pl.BlockSpec((1, d), lambda i: (0, 0)),
                  pl.BlockSpec((RB, 1), lambda i: (i, 0))],
        out_specs=pl.BlockSpec((nseg, d), lambda i: (0, 0)),
        out_shape=jax.ShapeDtypeStruct((nseg, d), F32),
    )(u, st, g, b, batch2d)


def _head_g(target, w_r2, vocab):
    bsz, seq = target.shape
    ok = w_r2.shape[1]

    def body(t_ref, w_ref, o_ref):
        v = pl.program_id(0)
        mask = (t_ref[...] == v).astype(F32)
        o_ref[0, :, :] = jnp.dot(mask, w_ref[...], preferred_element_type=F32, precision=HI)

    return pl.pallas_call(
        body,
        grid=(vocab,),
        in_specs=[pl.BlockSpec((bsz, seq), lambda v: (0, 0)),
                  pl.BlockSpec((seq, ok), lambda v: (0, 0))],
        out_specs=pl.BlockSpec((1, bsz, ok), lambda v: (v, 0, 0)),
        out_shape=jax.ShapeDtypeStruct((vocab, bsz, ok), F32),
    )(target, w_r2)


def _head_m(t_stack, w3t):
    kk, vocab, hh = t_stack.shape
    on = w3t.shape[1]
    nb = on // 1024

    def body(t_ref, w_ref, o_ref):
        o_ref[0, :, :] = jnp.dot(t_ref[0, :, :], w_ref[...],
                                 preferred_element_type=F32, precision=HI)

    return pl.pallas_call(
        body,
        grid=(kk, nb),
        in_specs=[pl.BlockSpec((1, vocab, hh), lambda k, nc: (k, 0, 0)),
                  pl.BlockSpec((hh, 1024), lambda k, nc: (0, nc))],
        out_specs=pl.BlockSpec((1, vocab, 1024), lambda k, nc: (k, 0, nc)),
        out_shape=jax.ShapeDtypeStruct((kk, vocab, on), F32),
    )(t_stack, w3t)


def _head_final(pool, g_all, m_all, xd_w, xd_b, xt_bias, fc1_w, fc1_b,
                fc2_w, fc2_b, cls_w, cls_b):
    bsz = pool.shape[0]
    ncls = cls_w.shape[1]

    def body(pool_ref, g_ref, m_ref, xdw_ref, xdb_ref, xtb_ref, f1w_ref,
             f1b_ref, f2w_ref, f2b_ref, cw_ref, cb_ref, o_ref):
        xt = jnp.dot(g_ref[...], m_ref[...],
                     preferred_element_type=F32, precision=HI) + xtb_ref[...]
        xd = jnp.maximum(jnp.dot(pool_ref[...], xdw_ref[...],
                                 preferred_element_type=F32, precision=HI) + xdb_ref[...], 0.0)
        xc = jnp.concatenate([xd, xt], axis=1)
        h1 = jnp.maximum(jnp.dot(xc, f1w_ref[...],
                                 preferred_element_type=F32, precision=HI) + f1b_ref[...], 0.0)
        h2 = jnp.maximum(jnp.dot(h1, f2w_ref[...],
                                 preferred_element_type=F32, precision=HI) + f2b_ref[...], 0.0)
        o_ref[...] = jnp.dot(h2, cw_ref[...],
                             preferred_element_type=F32, precision=HI) + cb_ref[...]

    return pl.pallas_call(
        body,
        out_shape=jax.ShapeDtypeStruct((bsz, ncls), F32),
    )(pool, g_all, m_all, xd_w, xd_b, xt_bias, fc1_w, fc1_b, fc2_w, fc2_b,
      cls_w, cls_b)


def kernel(x, edge_index, batch, target, emb_table, gin_params, bn_params,
           fc_params):
    n, fd = x.shape
    e = edge_index.shape[1]
    d = gin_params[0][0].shape[1]
    bsz, seq = target.shape
    vocab, emb = emb_table.shape
    odim = fc_params['fc1_xd_w'].shape[1]

    per_w = -(-e // (NC * NS * CHUNK * KS)) * (CHUNK * KS)
    e_pad = per_w * NC * NS
    n_pad = -(-(n + 1) // (NS * CHUNK)) * (NS * CHUNK)
    src = edge_index[0]
    dst = edge_index[1]
    pad = e_pad - e
    src_p = jnp.concatenate([src, jnp.zeros((pad,), jnp.int32)])
    dst_p = jnp.concatenate([dst, jnp.full((pad,), n, jnp.int32)])
    ei_p = jnp.stack([src_p.reshape(e_pad // CHUNK, CHUNK),
                      dst_p.reshape(e_pad // CHUNK, CHUNK)], axis=1)

    wa0, ba0, wb0, bb0 = gin_params[0]
    da = 32
    dt = 16
    x0 = x[:, :da]
    x1 = x[:, da:2 * da]
    xt = jnp.concatenate(
        [x[:, 2 * da:], jnp.zeros((n, dt - (fd - 2 * da) - 1), F32),
         jnp.ones((n, 1), F32)], axis=1)
    aa, ab = _sc_agg_fs(x0, x1, ei_p, n_pad, e_pad)
    t0, t1 = _sc_agg(xt, ei_p, n_pad, e_pad)
    u_pre, st = _mlp0(x, aa, ab, t0, t1, wa0, ba0.reshape(1, d), wb0,
                      bb0.reshape(1, d), fd)
    deg = (t0 + t1)[:, dt - 1:dt]
    for i in range(1, len(gin_params)):
        g, b = bn_params[i - 1]
        a0, a1 = _sc_agg(u_pre, ei_p, n_pad, e_pad)
        wa, ba, wb, bb = gin_params[i]
        u_pre, st = _mlp_fold(u_pre, st, g.reshape(1, d), b.reshape(1, d),
                              a0, a1, deg, wa, ba.reshape(1, d), wb,
                              bb.reshape(1, d))
    g, b = bn_params[-1]
    pool = _bn_pool(u_pre, st, g.reshape(1, d), b.reshape(1, d),
                    batch.reshape(n, 1), bsz)

    conv_w = fc_params['conv_w']
    osz, _, ksz = conv_w.shape
    hout = emb - ksz + 1
    w_r2 = conv_w.transpose(1, 2, 0).reshape(seq, ksz * osz)
    w3t = (fc_params['fc1_xt_w'].reshape(osz, hout, odim)
           .transpose(1, 0, 2).reshape(hout, osz * odim))
    t_stack = jnp.stack([emb_table[:, k:k + hout] for k in range(ksz)])
    xt_bias = (fc_params['conv_b'] @ fc_params['fc1_xt_w']
               .reshape(osz, hout, odim).sum(axis=1)
               + fc_params['fc1_xt_b']).reshape(1, odim)

    gt = _head_g(target, w_r2, vocab)
    g_all = gt.transpose(1, 0, 2).reshape(bsz, vocab * ksz * osz)
    m = _head_m(t_stack, w3t)
    m_all = (m.reshape(ksz, vocab, osz, odim).transpose(1, 0, 2, 3)
             .reshape(vocab * ksz * osz, odim))

    return _head_final(
        pool, g_all, m_all,
        fc_params['fc1_xd_w'], fc_params['fc1_xd_b'].reshape(1, odim),
        xt_bias,
        fc_params['fc1_w'], fc_params['fc1_b'].reshape(1, -1),
        fc_params['fc2_w'], fc_params['fc2_b'].reshape(1, -1),
        fc_params['cls_w'], fc_params['cls_b'].reshape(1, -1))
